# baseline (speedup 1.0000x reference)
import jax
import jax.numpy as jnp
from jax.experimental import pallas as pl


_CHUNK = 512


def _split_hi_lo(v):
    hi = v.astype(jnp.bfloat16)
    lo = (v - hi.astype(jnp.float32)).astype(jnp.bfloat16)
    return hi, lo


def _chamfer_kernel(a_ref, bm_ref, out_x_ref, out_y_ref):
    n = bm_ref.shape[1]
    d = bm_ref.shape[0] - 2
    n_chunks = n // _CHUNK

    a_hi, a_lo = _split_hi_lo(a_ref[...])
    b_hi, b_lo = _split_hi_lo(bm_ref[...])
    aa = jnp.concatenate(
        [
            a_hi,
            a_hi[:, 0:d],
            a_hi[:, d + 1 : d + 2],
            a_lo[:, 0 : d + 1],
            a_lo[:, 0:d],
        ],
        axis=1,
    )
    bb = jnp.concatenate(
        [
            b_hi,
            b_lo[0:d, :],
            b_lo[d + 1 : d + 2, :],
            b_hi[0 : d + 1, :],
            b_lo[0:d, :],
        ],
        axis=0,
    )

    ymin = jnp.full((n,), jnp.inf, dtype=jnp.float32)
    for i in range(n_chunks):
        a_chunk = aa[i * _CHUNK : (i + 1) * _CHUNK, :]
        t = jax.lax.dot_general(
            a_chunk,
            bb,
            (((1,), (0,)), ((), ())),
            preferred_element_type=jnp.float32,
        )
        out_x_ref[pl.ds(i * _CHUNK, _CHUNK), :] = jnp.min(t, axis=1, keepdims=True)
        ymin = jnp.minimum(ymin, jnp.min(t, axis=0))
    out_y_ref[0, :] = ymin


def kernel(x, y):
    b, d, n = x.shape
    f32 = jnp.float32
    k = d + 2

    xt = jnp.transpose(x, (0, 2, 1))
    nx = jnp.sum(xt * xt, axis=2, keepdims=True)
    a_full = jnp.concatenate(
        [-2.0 * xt, nx, jnp.ones((b, n, 1), f32)], axis=2
    )

    ny = jnp.sum(y * y, axis=1, keepdims=True)
    bm_full = jnp.concatenate(
        [y, jnp.ones((b, 1, n), f32), ny], axis=1
    )

    out_x, out_y = pl.pallas_call(
        _chamfer_kernel,
        grid=(b,),
        in_specs=[
            pl.BlockSpec((None, n, k), lambda i: (i, 0, 0)),
            pl.BlockSpec((None, k, n), lambda i: (i, 0, 0)),
        ],
        out_specs=[
            pl.BlockSpec((None, n, 1), lambda i: (i, 0, 0)),
            pl.BlockSpec((None, 1, n), lambda i: (i, 0, 0)),
        ],
        out_shape=[
            jax.ShapeDtypeStruct((b, n, 1), f32),
            jax.ShapeDtypeStruct((b, 1, n), f32),
        ],
    )(a_full, bm_full)

    return jnp.mean(out_x) + jnp.mean(out_y)


# speedup vs baseline: 1.0554x; 1.0554x over previous
import jax
import jax.numpy as jnp
from jax.experimental import pallas as pl


_CHUNK = 512


def _split_hi_lo(v):
    hi = v.astype(jnp.bfloat16)
    lo = (v - hi.astype(jnp.float32)).astype(jnp.bfloat16)
    return hi, lo


def _chamfer_kernel(a_ref, bm_ref, out_x_ref, out_y_ref):
    n = bm_ref.shape[1]
    d = bm_ref.shape[0] - 2
    n_chunks = n // _CHUNK

    a_hi, a_lo = _split_hi_lo(a_ref[...])
    b_hi, b_lo = _split_hi_lo(bm_ref[...])
    aa = jnp.concatenate(
        [
            a_hi,
            a_hi[:, 0:d],
            a_hi[:, d + 1 : d + 2],
            a_lo[:, 0 : d + 1],
            a_lo[:, 0:d],
        ],
        axis=1,
    )
    bb = jnp.concatenate(
        [
            b_hi,
            b_lo[0:d, :],
            b_lo[d + 1 : d + 2, :],
            b_hi[0 : d + 1, :],
            b_lo[0:d, :],
        ],
        axis=0,
    )

    ymin = jnp.full((n,), jnp.inf, dtype=jnp.float32)
    for i in range(n_chunks):
        a_chunk = aa[i * _CHUNK : (i + 1) * _CHUNK, :]
        t = jax.lax.dot_general(
            a_chunk,
            bb,
            (((1,), (0,)), ((), ())),
            preferred_element_type=jnp.float32,
        )
        out_x_ref[0, pl.ds(i * _CHUNK, _CHUNK)] = jnp.min(t, axis=1)
        ymin = jnp.minimum(ymin, jnp.min(t, axis=0))
    out_y_ref[0, :] = ymin


def kernel(x, y):
    b, d, n = x.shape
    f32 = jnp.float32
    k = d + 2

    xt = jnp.transpose(x, (0, 2, 1))
    nx = jnp.sum(xt * xt, axis=2, keepdims=True)
    a_full = jnp.concatenate(
        [-2.0 * xt, nx, jnp.ones((b, n, 1), f32)], axis=2
    )

    ny = jnp.sum(y * y, axis=1, keepdims=True)
    bm_full = jnp.concatenate(
        [y, jnp.ones((b, 1, n), f32), ny], axis=1
    )

    out_x, out_y = pl.pallas_call(
        _chamfer_kernel,
        grid=(b,),
        in_specs=[
            pl.BlockSpec((None, n, k), lambda i: (i, 0, 0)),
            pl.BlockSpec((None, k, n), lambda i: (i, 0, 0)),
        ],
        out_specs=[
            pl.BlockSpec((None, 1, n), lambda i: (i, 0, 0)),
            pl.BlockSpec((None, 1, n), lambda i: (i, 0, 0)),
        ],
        out_shape=[
            jax.ShapeDtypeStruct((b, 1, n), f32),
            jax.ShapeDtypeStruct((b, 1, n), f32),
        ],
    )(a_full, bm_full)

    return jnp.mean(out_x) + jnp.mean(out_y)


# speedup vs baseline: 1.1025x; 1.0446x over previous
import jax
import jax.numpy as jnp
from jax.experimental import pallas as pl


_CHUNK = 512


def _split_hi_lo(v):
    hi = v.astype(jnp.bfloat16)
    lo = (v - hi.astype(jnp.float32)).astype(jnp.bfloat16)
    return hi, lo


def _chamfer_kernel(a_ref, bm_ref, out_x_ref, out_y_ref):
    n = bm_ref.shape[1]
    d = bm_ref.shape[0] - 2
    n_chunks = n // _CHUNK

    a_hi, a_lo = _split_hi_lo(a_ref[...])
    b_hi, b_lo = _split_hi_lo(bm_ref[...])
    aa = jnp.concatenate(
        [
            a_hi,
            a_hi[:, 0:d],
            a_hi[:, d + 1 : d + 2],
            a_lo[:, 0 : d + 1],
            a_lo[:, 0:d],
        ],
        axis=1,
    )
    bb = jnp.concatenate(
        [
            b_hi,
            b_lo[0:d, :],
            b_lo[d + 1 : d + 2, :],
            b_hi[0 : d + 1, :],
            b_lo[0:d, :],
        ],
        axis=0,
    )

    ymin = jnp.full((1, n), jnp.inf, dtype=jnp.float32)
    mnacc = jnp.zeros((_CHUNK, 1), dtype=jnp.float32)
    for i in range(n_chunks):
        a_chunk = aa[i * _CHUNK : (i + 1) * _CHUNK, :]
        t = jax.lax.dot_general(
            a_chunk,
            bb,
            (((1,), (0,)), ((), ())),
            preferred_element_type=jnp.float32,
        )
        mnacc = mnacc + jnp.min(t, axis=1, keepdims=True)
        ymin = jnp.minimum(ymin, jnp.min(t, axis=0, keepdims=True))
    out_x_ref[...] = jnp.sum(mnacc, keepdims=True)
    out_y_ref[...] = jnp.sum(ymin, keepdims=True)


def kernel(x, y):
    b, d, n = x.shape
    f32 = jnp.float32
    k = d + 2

    xt = jnp.transpose(x, (0, 2, 1))
    nx = jnp.sum(xt * xt, axis=2, keepdims=True)
    a_full = jnp.concatenate(
        [-2.0 * xt, nx, jnp.ones((b, n, 1), f32)], axis=2
    )

    ny = jnp.sum(y * y, axis=1, keepdims=True)
    bm_full = jnp.concatenate(
        [y, jnp.ones((b, 1, n), f32), ny], axis=1
    )

    out_x, out_y = pl.pallas_call(
        _chamfer_kernel,
        grid=(b,),
        in_specs=[
            pl.BlockSpec((None, n, k), lambda i: (i, 0, 0)),
            pl.BlockSpec((None, k, n), lambda i: (i, 0, 0)),
        ],
        out_specs=[
            pl.BlockSpec((None, 1, 1), lambda i: (i, 0, 0)),
            pl.BlockSpec((None, 1, 1), lambda i: (i, 0, 0)),
        ],
        out_shape=[
            jax.ShapeDtypeStruct((b, 1, 1), f32),
            jax.ShapeDtypeStruct((b, 1, 1), f32),
        ],
    )(a_full, bm_full)

    return (jnp.sum(out_x) + jnp.sum(out_y)) / (b * n)


# speedup vs baseline: 1.3001x; 1.1793x over previous
import jax
import jax.numpy as jnp
from jax.experimental import pallas as pl


_CHUNK = 512


def _split_hi_lo(v):
    hi = v.astype(jnp.bfloat16)
    lo = (v - hi.astype(jnp.float32)).astype(jnp.bfloat16)
    return hi, lo


def _chamfer_kernel(x_ref, y_ref, out_x_ref, out_y_ref):
    d, n = x_ref.shape
    n_chunks = n // _CHUNK

    xv = x_ref[...]
    yv = y_ref[...]
    nx = jnp.sum(xv * xv, axis=0, keepdims=True)
    at_full = jnp.concatenate(
        [-2.0 * xv, nx, jnp.ones((1, n), jnp.float32)], axis=0
    )
    ny = jnp.sum(yv * yv, axis=0, keepdims=True)
    bt_full = jnp.concatenate(
        [yv, jnp.ones((1, n), jnp.float32), ny], axis=0
    )

    a_hi, a_lo = _split_hi_lo(at_full)
    b_hi, b_lo = _split_hi_lo(bt_full)
    aat = jnp.concatenate(
        [
            a_hi,
            a_hi[0:d, :],
            a_hi[d + 1 : d + 2, :],
            a_lo[0 : d + 1, :],
            a_lo[0:d, :],
        ],
        axis=0,
    )
    bbt = jnp.concatenate(
        [
            b_hi,
            b_lo[0:d, :],
            b_lo[d + 1 : d + 2, :],
            b_hi[0 : d + 1, :],
            b_lo[0:d, :],
        ],
        axis=0,
    )

    ymin = jnp.full((1, n), jnp.inf, dtype=jnp.float32)
    mnacc = jnp.zeros((_CHUNK, 1), dtype=jnp.float32)
    for i in range(n_chunks):
        a_chunk = aat[:, i * _CHUNK : (i + 1) * _CHUNK]
        t = jax.lax.dot_general(
            a_chunk,
            bbt,
            (((0,), (0,)), ((), ())),
            preferred_element_type=jnp.float32,
        )
        mnacc = mnacc + jnp.min(t, axis=1, keepdims=True)
        ymin = jnp.minimum(ymin, jnp.min(t, axis=0, keepdims=True))
    out_x_ref[...] = jnp.sum(mnacc, keepdims=True)
    out_y_ref[...] = jnp.sum(ymin, keepdims=True)


def kernel(x, y):
    b, d, n = x.shape
    f32 = jnp.float32

    out_x, out_y = pl.pallas_call(
        _chamfer_kernel,
        grid=(b,),
        in_specs=[
            pl.BlockSpec((None, d, n), lambda i: (i, 0, 0)),
            pl.BlockSpec((None, d, n), lambda i: (i, 0, 0)),
        ],
        out_specs=[
            pl.BlockSpec((None, 1, 1), lambda i: (i, 0, 0)),
            pl.BlockSpec((None, 1, 1), lambda i: (i, 0, 0)),
        ],
        out_shape=[
            jax.ShapeDtypeStruct((b, 1, 1), f32),
            jax.ShapeDtypeStruct((b, 1, 1), f32),
        ],
    )(x, y)

    return (jnp.sum(out_x) + jnp.sum(out_y)) / (b * n)


# speedup vs baseline: 1.3016x; 1.0011x over previous
import jax
import jax.numpy as jnp
from jax.experimental import pallas as pl


_CHUNK = 512


def _split_hi_lo(v):
    hi = v.astype(jnp.bfloat16)
    lo = (v - hi.astype(jnp.float32)).astype(jnp.bfloat16)
    return hi, lo


def _chamfer_kernel(x_ref, y_ref, out_x_ref, out_y_ref):
    d, n = x_ref.shape
    n_chunks = n // _CHUNK

    xv = x_ref[...]
    yv = y_ref[...]
    nx = jnp.sum(xv * xv, axis=0, keepdims=True)
    at_full = jnp.concatenate(
        [-2.0 * xv, nx, jnp.ones((1, n), jnp.float32)], axis=0
    )
    ny = jnp.sum(yv * yv, axis=0, keepdims=True)
    bt_full = jnp.concatenate(
        [yv, jnp.ones((1, n), jnp.float32), ny], axis=0
    )

    a_hi, a_lo = _split_hi_lo(at_full)
    b_hi, b_lo = _split_hi_lo(bt_full)
    aat = jnp.concatenate(
        [
            a_hi,
            a_hi[0:d, :],
            a_hi[d + 1 : d + 2, :],
            a_lo[0 : d + 1, :],
            a_lo[0:d, :],
        ],
        axis=0,
    )
    bbt = jnp.concatenate(
        [
            b_hi,
            b_lo[0:d, :],
            b_lo[d + 1 : d + 2, :],
            b_hi[0 : d + 1, :],
            b_lo[0:d, :],
        ],
        axis=0,
    )

    def chunk_dist(i):
        a_chunk = aat[:, i * _CHUNK : (i + 1) * _CHUNK]
        return jax.lax.dot_general(
            a_chunk,
            bbt,
            (((0,), (0,)), ((), ())),
            preferred_element_type=jnp.float32,
        )

    ymin = jnp.full((1, n), jnp.inf, dtype=jnp.float32)
    mnacc = jnp.zeros((_CHUNK, 1), dtype=jnp.float32)
    t_cur = chunk_dist(0)
    for i in range(n_chunks):
        t_next = chunk_dist(i + 1) if i + 1 < n_chunks else None
        mnacc = mnacc + jnp.min(t_cur, axis=1, keepdims=True)
        ymin = jnp.minimum(ymin, jnp.min(t_cur, axis=0, keepdims=True))
        t_cur = t_next
    out_x_ref[...] = jnp.sum(mnacc, keepdims=True)
    out_y_ref[...] = jnp.sum(ymin, keepdims=True)


def kernel(x, y):
    b, d, n = x.shape
    f32 = jnp.float32

    out_x, out_y = pl.pallas_call(
        _chamfer_kernel,
        grid=(b,),
        in_specs=[
            pl.BlockSpec((None, d, n), lambda i: (i, 0, 0)),
            pl.BlockSpec((None, d, n), lambda i: (i, 0, 0)),
        ],
        out_specs=[
            pl.BlockSpec((None, 1, 1), lambda i: (i, 0, 0)),
            pl.BlockSpec((None, 1, 1), lambda i: (i, 0, 0)),
        ],
        out_shape=[
            jax.ShapeDtypeStruct((b, 1, 1), f32),
            jax.ShapeDtypeStruct((b, 1, 1), f32),
        ],
    )(x, y)

    return (jnp.sum(out_x) + jnp.sum(out_y)) / (b * n)


# speedup vs baseline: 1.3033x; 1.0013x over previous
import jax
import jax.numpy as jnp
from jax.experimental import pallas as pl


_CHUNK = 1024


def _split_hi_lo(v):
    hi = v.astype(jnp.bfloat16)
    lo = (v - hi.astype(jnp.float32)).astype(jnp.bfloat16)
    return hi, lo


def _chamfer_kernel(x_ref, y_ref, out_x_ref, out_y_ref):
    d, n = x_ref.shape
    n_chunks = n // _CHUNK

    xv = x_ref[...]
    yv = y_ref[...]
    nx = jnp.sum(xv * xv, axis=0, keepdims=True)
    at_full = jnp.concatenate(
        [-2.0 * xv, nx, jnp.ones((1, n), jnp.float32)], axis=0
    )
    ny = jnp.sum(yv * yv, axis=0, keepdims=True)
    bt_full = jnp.concatenate(
        [yv, jnp.ones((1, n), jnp.float32), ny], axis=0
    )

    a_hi, a_lo = _split_hi_lo(at_full)
    b_hi, b_lo = _split_hi_lo(bt_full)
    aat = jnp.concatenate(
        [
            a_hi,
            a_hi[0:d, :],
            a_hi[d + 1 : d + 2, :],
            a_lo[0 : d + 1, :],
            a_lo[0:d, :],
        ],
        axis=0,
    )
    bbt = jnp.concatenate(
        [
            b_hi,
            b_lo[0:d, :],
            b_lo[d + 1 : d + 2, :],
            b_hi[0 : d + 1, :],
            b_lo[0:d, :],
        ],
        axis=0,
    )

    def chunk_dist(i):
        a_chunk = aat[:, i * _CHUNK : (i + 1) * _CHUNK]
        return jax.lax.dot_general(
            a_chunk,
            bbt,
            (((0,), (0,)), ((), ())),
            preferred_element_type=jnp.float32,
        )

    ymin = jnp.full((1, n), jnp.inf, dtype=jnp.float32)
    mnacc = jnp.zeros((_CHUNK, 1), dtype=jnp.float32)
    t_cur = chunk_dist(0)
    for i in range(n_chunks):
        t_next = chunk_dist(i + 1) if i + 1 < n_chunks else None
        mnacc = mnacc + jnp.min(t_cur, axis=1, keepdims=True)
        ymin = jnp.minimum(ymin, jnp.min(t_cur, axis=0, keepdims=True))
        t_cur = t_next
    out_x_ref[...] = jnp.sum(mnacc, keepdims=True)
    out_y_ref[...] = jnp.sum(ymin, keepdims=True)


def kernel(x, y):
    b, d, n = x.shape
    f32 = jnp.float32

    out_x, out_y = pl.pallas_call(
        _chamfer_kernel,
        grid=(b,),
        in_specs=[
            pl.BlockSpec((None, d, n), lambda i: (i, 0, 0)),
            pl.BlockSpec((None, d, n), lambda i: (i, 0, 0)),
        ],
        out_specs=[
            pl.BlockSpec((None, 1, 1), lambda i: (i, 0, 0)),
            pl.BlockSpec((None, 1, 1), lambda i: (i, 0, 0)),
        ],
        out_shape=[
            jax.ShapeDtypeStruct((b, 1, 1), f32),
            jax.ShapeDtypeStruct((b, 1, 1), f32),
        ],
    )(x, y)

    return (jnp.sum(out_x) + jnp.sum(out_y)) / (b * n)
